# Initial kernel scaffold; baseline (speedup 1.0000x reference)
import jax
import jax.numpy as jnp
from jax.experimental import pallas as pl


def kernel(x, edge_index, W1a, b1a, W1b, b1b, W2a, b2a, W2b, b2b):
    raise NotImplementedError("write your pallas kernel here")


# speedup vs baseline: 7.2622x; 7.2622x over previous
import functools

import jax
import jax.numpy as jnp
from jax import lax
from jax.experimental import pallas as pl
from jax.experimental.pallas import tpu as pltpu
from jax.experimental.pallas import tpu_sc as plsc

N = 10000
E = 320000
D = 128
NC = 2
NS = 16
NW = NC * NS
EPW = E // NW
CHUNK = 80
NCHUNK = EPW // CHUNK
ROWS_PER_TILE = 640
NPAD = NS * ROWS_PER_TILE


def _make_agg():
  mesh = plsc.VectorSubcoreMesh(core_axis_name="c", subcore_axis_name="s")

  @functools.partial(
      pl.kernel,
      out_type=jax.ShapeDtypeStruct((NC, NPAD, D), jnp.float32),
      mesh=mesh,
      scratch_types=[
          pltpu.VMEM_SHARED((NPAD, D), jnp.float32),
          pltpu.VMEM((NCHUNK, CHUNK), jnp.int32),
          pltpu.VMEM((NCHUNK, CHUNK), jnp.int32),
          pltpu.VMEM((CHUNK, D), jnp.float32),
      ],
  )
  def agg(x_hbm, src_hbm, dst_hbm, zeros_hbm, out_hbm, acc, src_v, dst_v,
          rows_v):
    c = lax.axis_index("c")
    s = lax.axis_index("s")
    wid = s * NC + c
    pltpu.sync_copy(zeros_hbm, acc.at[pl.ds(s * ROWS_PER_TILE, ROWS_PER_TILE)])
    pltpu.sync_copy(src_hbm.at[wid], src_v)
    pltpu.sync_copy(dst_hbm.at[wid], dst_v)
    plsc.subcore_barrier()

    def body(i, carry):
      pltpu.sync_copy(x_hbm.at[src_v.at[i]], rows_v)
      pltpu.sync_copy(rows_v, acc.at[dst_v.at[i]], add=True)
      return carry

    lax.fori_loop(0, NCHUNK, body, 0)
    plsc.subcore_barrier()
    pltpu.sync_copy(
        acc.at[pl.ds(s * ROWS_PER_TILE, ROWS_PER_TILE)],
        out_hbm.at[c, pl.ds(s * ROWS_PER_TILE, ROWS_PER_TILE)])

  return agg


_BLK = 1000


def _mlp_body(x_ref, p_ref, wa_ref, ba_ref, wb_ref, bb_ref, o_ref):
  h = x_ref[...] + p_ref[0] + p_ref[1]
  h = jnp.dot(h, wa_ref[...], preferred_element_type=jnp.float32) + ba_ref[...]
  h = jnp.maximum(h, 0.0)
  o_ref[...] = (
      jnp.dot(h, wb_ref[...], preferred_element_type=jnp.float32) + bb_ref[...])


def _mlp(x, p, Wa, ba, Wb, bb):
  return pl.pallas_call(
      _mlp_body,
      grid=(N // _BLK,),
      in_specs=[
          pl.BlockSpec((_BLK, D), lambda i: (i, 0)),
          pl.BlockSpec((NC, _BLK, D), lambda i: (0, i, 0)),
          pl.BlockSpec((D, D), lambda i: (0, 0)),
          pl.BlockSpec((1, D), lambda i: (0, 0)),
          pl.BlockSpec((D, D), lambda i: (0, 0)),
          pl.BlockSpec((1, D), lambda i: (0, 0)),
      ],
      out_specs=pl.BlockSpec((_BLK, D), lambda i: (i, 0)),
      out_shape=jax.ShapeDtypeStruct((N, D), jnp.float32),
  )(x, p, Wa, ba, Wb, bb)


def kernel(x, edge_index, W1a, b1a, W1b, b1b, W2a, b2a, W2b, b2b):
  src = edge_index[0].reshape(NW, NCHUNK, CHUNK)
  dst = edge_index[1].reshape(NW, NCHUNK, CHUNK)
  zeros = jnp.zeros((ROWS_PER_TILE, D), jnp.float32)
  agg = _make_agg()
  p1 = agg(x, src, dst, zeros)
  h1 = _mlp(x, p1, W1a, b1a.reshape(1, D), W1b, b1b.reshape(1, D))
  p2 = agg(h1, src, dst, zeros)
  h2 = _mlp(h1, p2, W2a, b2a.reshape(1, D), W2b, b2b.reshape(1, D))
  return h2


# speedup vs baseline: 12.0543x; 1.6599x over previous
import functools

import jax
import jax.numpy as jnp
from jax import lax
from jax.experimental import pallas as pl
from jax.experimental.pallas import tpu as pltpu
from jax.experimental.pallas import tpu_sc as plsc

N = 10000
E = 320000
D = 128
NC = 2
NS = 16
NW = NC * NS
EPW = E // NW
CHUNK = 100
NCHUNK = EPW // CHUNK
NBUF = 2
ROWS_PER_TILE = 640
NPAD = NS * ROWS_PER_TILE


def _make_agg():
  mesh = plsc.VectorSubcoreMesh(core_axis_name="c", subcore_axis_name="s")

  @functools.partial(
      pl.kernel,
      out_type=jax.ShapeDtypeStruct((NC, NPAD, D), jnp.float32),
      mesh=mesh,
      compiler_params=pltpu.CompilerParams(use_tc_tiling_on_sc=False),
      scratch_types=[
          pltpu.VMEM_SHARED((NPAD, D), jnp.float32),
          pltpu.VMEM((NCHUNK, CHUNK), jnp.int32),
          pltpu.VMEM((NCHUNK, CHUNK), jnp.int32),
          pltpu.VMEM((NBUF, CHUNK, D), jnp.float32),
          [pltpu.SemaphoreType.DMA] * NBUF,
      ],
  )
  def agg(x_hbm, src_hbm, dst_hbm, zeros_hbm, out_hbm, acc, src_v, dst_v,
          rows_v, sems):
    c = lax.axis_index("c")
    s = lax.axis_index("s")
    wid = s * NC + c
    pltpu.sync_copy(zeros_hbm, acc.at[pl.ds(s * ROWS_PER_TILE, ROWS_PER_TILE)])
    pltpu.sync_copy(src_hbm.at[wid], src_v)
    pltpu.sync_copy(dst_hbm.at[wid], dst_v)
    plsc.subcore_barrier()

    for b in range(NBUF):
      pltpu.async_copy(x_hbm.at[src_v.at[b]], rows_v.at[b], sems[b])

    @pl.loop(0, NCHUNK, step=NBUF)
    def _(g):
      for b in range(NBUF):
        pltpu.make_async_copy(
            x_hbm.at[src_v.at[g + b]], rows_v.at[b], sems[b]).wait()
        pltpu.sync_copy(rows_v.at[b], acc.at[dst_v.at[g + b]], add=True)

        @pl.when(g < NCHUNK - NBUF)
        def _():
          pltpu.async_copy(
              x_hbm.at[src_v.at[g + b + NBUF]], rows_v.at[b], sems[b])

    plsc.subcore_barrier()
    pltpu.sync_copy(
        acc.at[pl.ds(s * ROWS_PER_TILE, ROWS_PER_TILE)],
        out_hbm.at[c, pl.ds(s * ROWS_PER_TILE, ROWS_PER_TILE)])

  return agg


_BLK = 1000


def _mlp_body(x_ref, p_ref, wa_ref, ba_ref, wb_ref, bb_ref, o_ref):
  h = x_ref[...] + p_ref[0] + p_ref[1]
  h = jnp.dot(h, wa_ref[...], preferred_element_type=jnp.float32) + ba_ref[...]
  h = jnp.maximum(h, 0.0)
  o_ref[...] = (
      jnp.dot(h, wb_ref[...], preferred_element_type=jnp.float32) + bb_ref[...])


def _mlp(x, p, Wa, ba, Wb, bb):
  return pl.pallas_call(
      _mlp_body,
      grid=(N // _BLK,),
      in_specs=[
          pl.BlockSpec((_BLK, D), lambda i: (i, 0)),
          pl.BlockSpec((NC, _BLK, D), lambda i: (0, i, 0)),
          pl.BlockSpec((D, D), lambda i: (0, 0)),
          pl.BlockSpec((1, D), lambda i: (0, 0)),
          pl.BlockSpec((D, D), lambda i: (0, 0)),
          pl.BlockSpec((1, D), lambda i: (0, 0)),
      ],
      out_specs=pl.BlockSpec((_BLK, D), lambda i: (i, 0)),
      out_shape=jax.ShapeDtypeStruct((N, D), jnp.float32),
  )(x, p, Wa, ba, Wb, bb)


def kernel(x, edge_index, W1a, b1a, W1b, b1b, W2a, b2a, W2b, b2b):
  src = edge_index[0].reshape(NW, NCHUNK, CHUNK)
  dst = edge_index[1].reshape(NW, NCHUNK, CHUNK)
  zeros = jnp.zeros((ROWS_PER_TILE, D), jnp.float32)
  agg = _make_agg()
  p1 = agg(x, src, dst, zeros)
  h1 = _mlp(x, p1, W1a, b1a.reshape(1, D), W1b, b1b.reshape(1, D))
  p2 = agg(h1, src, dst, zeros)
  h2 = _mlp(h1, p2, W2a, b2a.reshape(1, D), W2b, b2b.reshape(1, D))
  return h2


# speedup vs baseline: 12.4831x; 1.0356x over previous
import functools

import jax
import jax.numpy as jnp
from jax import lax
from jax.experimental import pallas as pl
from jax.experimental.pallas import tpu as pltpu
from jax.experimental.pallas import tpu_sc as plsc

N = 10000
E = 320000
D = 128
NC = 2
NS = 16
NW = NC * NS
EPW = E // NW
CHUNK = 100
NCHUNK = EPW // CHUNK
NBUF = 2
ROWS_PER_TILE = 640
NPAD = NS * ROWS_PER_TILE


def _make_agg():
  mesh = plsc.VectorSubcoreMesh(core_axis_name="c", subcore_axis_name="s")

  @functools.partial(
      pl.kernel,
      out_type=jax.ShapeDtypeStruct((NC, NPAD, D), jnp.float32),
      mesh=mesh,
      compiler_params=pltpu.CompilerParams(use_tc_tiling_on_sc=False),
      scratch_types=[
          pltpu.VMEM_SHARED((NPAD, D), jnp.float32),
          pltpu.VMEM((NCHUNK, CHUNK), jnp.int32),
          pltpu.VMEM((NCHUNK, CHUNK), jnp.int32),
          pltpu.VMEM((NBUF, CHUNK, D), jnp.float32),
          [pltpu.SemaphoreType.DMA] * NBUF,
          pltpu.SemaphoreType.DMA,
      ],
  )
  def agg(x_hbm, src_hbm, dst_hbm, zeros_hbm, out_hbm, acc, src_v, dst_v,
          rows_v, sems, zsem):
    c = lax.axis_index("c")
    s = lax.axis_index("s")
    wid = s * NC + c
    pltpu.sync_copy(src_hbm.at[wid], src_v)
    zero_cp = pltpu.async_copy(
        zeros_hbm, acc.at[pl.ds(s * ROWS_PER_TILE, ROWS_PER_TILE)], zsem)
    for b in range(NBUF):
      pltpu.async_copy(x_hbm.at[src_v.at[b]], rows_v.at[b], sems[b])
    pltpu.sync_copy(dst_hbm.at[wid], dst_v)
    zero_cp.wait()
    plsc.subcore_barrier()

    @pl.loop(0, NCHUNK, step=NBUF)
    def _(g):
      for b in range(NBUF):
        pltpu.make_async_copy(
            x_hbm.at[src_v.at[g + b]], rows_v.at[b], sems[b]).wait()
        pltpu.sync_copy(rows_v.at[b], acc.at[dst_v.at[g + b]], add=True)

        @pl.when(g < NCHUNK - NBUF)
        def _():
          pltpu.async_copy(
              x_hbm.at[src_v.at[g + b + NBUF]], rows_v.at[b], sems[b])

    plsc.subcore_barrier()
    pltpu.sync_copy(
        acc.at[pl.ds(s * ROWS_PER_TILE, ROWS_PER_TILE)],
        out_hbm.at[c, pl.ds(s * ROWS_PER_TILE, ROWS_PER_TILE)])

  return agg


_BLK = 2000


def _mlp_body(x_ref, p_ref, wa_ref, ba_ref, wb_ref, bb_ref, o_ref):
  h = x_ref[...] + p_ref[0] + p_ref[1]
  h = jnp.dot(h, wa_ref[...], preferred_element_type=jnp.float32) + ba_ref[...]
  h = jnp.maximum(h, 0.0)
  o_ref[...] = (
      jnp.dot(h, wb_ref[...], preferred_element_type=jnp.float32) + bb_ref[...])


def _mlp(x, p, Wa, ba, Wb, bb):
  return pl.pallas_call(
      _mlp_body,
      grid=(N // _BLK,),
      in_specs=[
          pl.BlockSpec((_BLK, D), lambda i: (i, 0)),
          pl.BlockSpec((NC, _BLK, D), lambda i: (0, i, 0)),
          pl.BlockSpec((D, D), lambda i: (0, 0)),
          pl.BlockSpec((1, D), lambda i: (0, 0)),
          pl.BlockSpec((D, D), lambda i: (0, 0)),
          pl.BlockSpec((1, D), lambda i: (0, 0)),
      ],
      out_specs=pl.BlockSpec((_BLK, D), lambda i: (i, 0)),
      out_shape=jax.ShapeDtypeStruct((N, D), jnp.float32),
  )(x, p, Wa, ba, Wb, bb)


def kernel(x, edge_index, W1a, b1a, W1b, b1b, W2a, b2a, W2b, b2b):
  src = edge_index[0].reshape(NW, NCHUNK, CHUNK)
  dst = edge_index[1].reshape(NW, NCHUNK, CHUNK)
  zeros = jnp.zeros((ROWS_PER_TILE, D), jnp.float32)
  agg = _make_agg()
  p1 = agg(x, src, dst, zeros)
  h1 = _mlp(x, p1, W1a, b1a.reshape(1, D), W1b, b1b.reshape(1, D))
  p2 = agg(h1, src, dst, zeros)
  h2 = _mlp(h1, p2, W2a, b2a.reshape(1, D), W2b, b2b.reshape(1, D))
  return h2
